# baseline (speedup 1.0000x reference)
import functools
import math

import jax
import jax.numpy as jnp
from jax import lax
from jax.experimental import pallas as pl
from jax.experimental.pallas import tpu as pltpu
from jax.experimental.pallas import tpu_sc as plsc

B, T, D = 4, 2048, 1024
NE = 32
NT = 7
ME = 32
THRESH = 0.4
ZBW = 0.15
MAX_TIME = 512

TT = 512
NROWS = B * T
NSEL = B * ME
TPB = T // TT

_NEG_INF = float("-inf")


def _heads_body(x_ref, z_ref, hl_ref, w1_ref, b1_ref, w2_ref, b2_ref,
                wp_ref, bp_ref, wn_ref, bn_ref,
                etl_ref, prev_ref, next_ref, nn_ref, zb_ref, ty_ref, hlp_ref,
                w1b_ref):
    @pl.when(pl.program_id(0) == 0)
    def _():
        w1b_ref[...] = w1_ref[...].astype(jnp.bfloat16)

    x = x_ref[...]
    xb = x.astype(jnp.bfloat16)
    h = jnp.dot(xb, w1b_ref[...], preferred_element_type=jnp.float32) + b1_ref[...]
    g = 0.5 * h * (lax.erf(h / math.sqrt(2.0)) + 1.0)
    etl = jnp.dot(g.astype(jnp.bfloat16), w2_ref[...].astype(jnp.bfloat16),
                  preferred_element_type=jnp.float32) + b2_ref[...]
    etl_ref[...] = etl
    prev_ref[...] = jnp.dot(xb, wp_ref[...].astype(jnp.bfloat16),
                            preferred_element_type=jnp.float32) + bp_ref[...]
    next_ref[...] = jnp.dot(xb, wn_ref[...].astype(jnp.bfloat16),
                            preferred_element_type=jnp.float32) + bn_ref[...]

    emax = jnp.max(etl, axis=1, keepdims=True)
    ee = jnp.exp(etl - emax)
    esum = jnp.sum(ee, axis=1, keepdims=True)
    nn = 1.0 - ee[:, :1] / esum
    it2 = lax.broadcasted_iota(jnp.int32, (TT, NT), 1)
    ty = jnp.min(jnp.where(etl == emax, it2, NT), axis=1, keepdims=True)
    zb = jnp.max(jnp.abs(z_ref[...]), axis=1, keepdims=True)

    nn_ref[...] = nn.T.reshape(1, 1, TT)
    zb_ref[...] = zb.T.reshape(1, 1, TT)
    ty_ref[...] = ty.T.reshape(1, 1, TT)

    hl = hl_ref[...]
    hlp_ref[...] = jnp.concatenate(
        [hl, jnp.zeros((TT, 128 - NE), jnp.float32)], axis=1)


def _heads(seq2d, z2d, hl2d, W1, b1, W2, b2, Wp, bp, Wn, bn):
    nt = NROWS // TT
    row = lambda w: pl.BlockSpec((TT, w), lambda t: (t, 0))
    vec = lambda: pl.BlockSpec((1, 1, TT), lambda t: (t // TPB, 0, t % TPB))
    full = lambda a: pl.BlockSpec(a.shape, lambda t: (0,) * a.ndim)
    f32 = jnp.float32
    return pl.pallas_call(
        _heads_body,
        grid=(nt,),
        in_specs=[row(D), row(NE), row(NE)]
        + [full(a) for a in (W1, b1, W2, b2, Wp, bp, Wn, bn)],
        out_specs=[row(NT), row(NE), row(NE), vec(), vec(), vec(), row(128)],
        out_shape=[
            jax.ShapeDtypeStruct((NROWS, NT), f32),
            jax.ShapeDtypeStruct((NROWS, NE), f32),
            jax.ShapeDtypeStruct((NROWS, NE), f32),
            jax.ShapeDtypeStruct((B, 1, T), f32),
            jax.ShapeDtypeStruct((B, 1, T), f32),
            jax.ShapeDtypeStruct((B, 1, T), jnp.int32),
            jax.ShapeDtypeStruct((NROWS, 128), f32),
        ],
        scratch_shapes=[pltpu.VMEM((D, D), jnp.bfloat16)],
    )(seq2d, z2d, hl2d, W1, b1, W2, b2, Wp, bp, Wn, bn)


def _select_body(nn_ref, zbr_ref, ty_ref, scores_ref, mask_ref, times_ref,
                 tids_ref, idxflat_ref, idxtime_ref, idxty_ref, idxmsk_ref):
    nn = nn_ref[...].reshape(B, T)
    zb = zbr_ref[...].reshape(B, T)
    ty = ty_ref[...].reshape(B, T)

    zmax = jnp.max(zb, axis=1, keepdims=True)
    scores = nn + ZBW * zb / jnp.maximum(zmax, 1.0)
    scores_ref[...] = scores

    iota_t = lax.broadcasted_iota(jnp.int32, (B, T), 1)

    key0 = jnp.where(scores >= THRESH, scores, _NEG_INF)

    def ext_step(_, carry):
        key, selmask, cnt = carry
        m = jnp.max(key, axis=1, keepdims=True)
        has = m > _NEG_INF
        hit = key == m
        idx = jnp.min(jnp.where(hit, iota_t, T), axis=1, keepdims=True)
        pick = (iota_t == idx) & has
        return (jnp.where(pick, _NEG_INF, key), selmask | pick.astype(jnp.int32),
                cnt + has.astype(jnp.int32))

    selmask0 = jnp.zeros((B, T), jnp.int32)
    cnt0 = jnp.zeros((B, 1), jnp.int32)
    _, selmask, n_eff = lax.fori_loop(0, ME, ext_step, (key0, selmask0, cnt0))

    ikey = jnp.where(selmask > 0, iota_t, T)
    ch_cols, ty_cols = [], []
    for _ in range(ME):
        idx = jnp.min(ikey, axis=1, keepdims=True)
        ch_cols.append(idx)
        ty_cols.append(jnp.max(jnp.where(iota_t == idx, ty, 0),
                               axis=1, keepdims=True))
        ikey = jnp.where(iota_t == idx, T, ikey)
    ch = jnp.concatenate(ch_cols, axis=1)
    tysel = jnp.concatenate(ty_cols, axis=1)

    empty = n_eff == 0
    slot = lax.broadcasted_iota(jnp.int32, (B, ME), 1)
    ch = jnp.where(empty, slot, ch)
    tysel = jnp.where(empty, ty[:, :ME], tysel)
    n = jnp.where(empty, ME, n_eff)
    mask = slot < n
    mask_i = mask.astype(jnp.int32)

    mask_ref[...] = mask
    times_ref[...] = jnp.where(mask, ch, 0)
    tids_ref[...] = jnp.where(mask, tysel, 0)

    def flat_row(a):
        return jnp.concatenate([a[b:b + 1, :] for b in range(B)], axis=1)

    bi = lax.broadcasted_iota(jnp.int32, (B, ME), 0)
    idxflat_ref[...] = flat_row(bi * T + jnp.minimum(ch, T - 1))
    idxtime_ref[...] = flat_row(jnp.clip(ch, 0, MAX_TIME - 1))
    idxty_ref[...] = flat_row(tysel)
    idxmsk_ref[...] = flat_row(mask_i)


def _select(nn, zb, ty):
    i32 = jnp.int32
    return pl.pallas_call(
        _select_body,
        out_shape=[
            jax.ShapeDtypeStruct((B, T), jnp.float32),
            jax.ShapeDtypeStruct((B, ME), jnp.bool_),
            jax.ShapeDtypeStruct((B, ME), i32),
            jax.ShapeDtypeStruct((B, ME), i32),
            jax.ShapeDtypeStruct((1, NSEL), i32),
            jax.ShapeDtypeStruct((1, NSEL), i32),
            jax.ShapeDtypeStruct((1, NSEL), i32),
            jax.ShapeDtypeStruct((1, NSEL), i32),
        ],
    )(nn, zb, ty)


_NC = 2
_NWORK = 16
_RPW = NSEL // _NWORK


def _sc_gather_body(seq_hbm, ss_hbm, hl_hbm, te_hbm, tye_hbm, mrow_hbm,
                    idxf_hbm, idxt_hbm, idxy_hbm, idxm_hbm,
                    seqo_hbm, sso_hbm, hlo_hbm, teo_hbm, tyeo_hbm, mrowo_hbm,
                    ia_v, ib_v, ic_v, buf1_v, buf2_v, sbuf_v, sem):
    wid = lax.axis_index("s") * _NC + lax.axis_index("c")
    j = lax.rem(wid, _NWORK)
    base = j * _RPW

    @pl.when(wid < _NWORK)
    def _():
        pltpu.sync_copy(idxf_hbm.at[0, pl.ds(base, _RPW)], ia_v)
        pltpu.sync_copy(idxy_hbm.at[0, pl.ds(base, _RPW)], ic_v)
        c1 = pltpu.async_copy(seq_hbm.at[ia_v], buf1_v, sem)
        c2 = pltpu.async_copy(hl_hbm.at[ia_v], sbuf_v, sem)
        c3 = pltpu.async_copy(tye_hbm.at[ic_v], buf2_v, sem)
        c1.wait()
        c2.wait()
        c3.wait()
        o1 = pltpu.async_copy(buf1_v, seqo_hbm.at[pl.ds(base, _RPW)], sem)
        o2 = pltpu.async_copy(sbuf_v, hlo_hbm.at[pl.ds(base, _RPW)], sem)
        o3 = pltpu.async_copy(buf2_v, tyeo_hbm.at[pl.ds(base, _RPW)], sem)
        o1.wait()
        o2.wait()
        o3.wait()

    @pl.when(wid >= _NWORK)
    def _():
        pltpu.sync_copy(idxf_hbm.at[0, pl.ds(base, _RPW)], ia_v)
        pltpu.sync_copy(idxt_hbm.at[0, pl.ds(base, _RPW)], ib_v)
        pltpu.sync_copy(idxm_hbm.at[0, pl.ds(base, _RPW)], ic_v)
        c1 = pltpu.async_copy(ss_hbm.at[ia_v], buf1_v, sem)
        c2 = pltpu.async_copy(te_hbm.at[ib_v], buf2_v, sem)
        c3 = pltpu.async_copy(mrow_hbm.at[ic_v], sbuf_v, sem)
        c1.wait()
        c2.wait()
        c3.wait()
        o1 = pltpu.async_copy(buf1_v, sso_hbm.at[pl.ds(base, _RPW)], sem)
        o2 = pltpu.async_copy(buf2_v, teo_hbm.at[pl.ds(base, _RPW)], sem)
        o3 = pltpu.async_copy(sbuf_v, mrowo_hbm.at[pl.ds(base, _RPW)], sem)
        o1.wait()
        o2.wait()
        o3.wait()


def _sc_gather(seq2d, ss2d, hlp, time_embed, type_embed, mrow_tbl,
               idx_flat, idx_time, idx_ty, idx_msk):
    f32 = jnp.float32
    mesh = plsc.VectorSubcoreMesh(core_axis_name="c", subcore_axis_name="s")
    k = pl.kernel(
        _sc_gather_body,
        out_type=[
            jax.ShapeDtypeStruct((NSEL, D), f32),
            jax.ShapeDtypeStruct((NSEL, D), f32),
            jax.ShapeDtypeStruct((NSEL, 128), f32),
            jax.ShapeDtypeStruct((NSEL, D), f32),
            jax.ShapeDtypeStruct((NSEL, D), f32),
            jax.ShapeDtypeStruct((NSEL, 128), f32),
        ],
        mesh=mesh,
        scratch_types=[
            pltpu.VMEM((_RPW,), jnp.int32),
            pltpu.VMEM((_RPW,), jnp.int32),
            pltpu.VMEM((_RPW,), jnp.int32),
            pltpu.VMEM((_RPW, D), f32),
            pltpu.VMEM((_RPW, D), f32),
            pltpu.VMEM((_RPW, 128), f32),
            pltpu.SemaphoreType.DMA,
        ],
    )
    return k(seq2d, ss2d, hlp, time_embed, type_embed, mrow_tbl,
             idx_flat, idx_time, idx_ty, idx_msk)


def _entries_body(seqs_ref, sss_ref, hls_ref, tes_ref, tyes_ref, mrows_ref,
                  we_ref, be_ref, out_ref):
    hl = hls_ref[:, :NE]
    hmax = jnp.max(hl, axis=1, keepdims=True)
    he = jnp.exp(hl - hmax)
    hp = he / jnp.sum(he, axis=1, keepdims=True)

    acc = jnp.dot(seqs_ref[...].astype(jnp.bfloat16),
                  we_ref[:D].astype(jnp.bfloat16),
                  preferred_element_type=jnp.float32)
    acc = acc + jnp.dot(sss_ref[...].astype(jnp.bfloat16),
                        we_ref[D:2 * D].astype(jnp.bfloat16),
                        preferred_element_type=jnp.float32)
    acc = acc + jnp.dot(hp.astype(jnp.bfloat16),
                        we_ref[2 * D:].astype(jnp.bfloat16),
                        preferred_element_type=jnp.float32)
    acc = acc + be_ref[...] + tyes_ref[...] + tes_ref[...]
    out_ref[...] = acc * mrows_ref[:, :1]


def _entries(seq_sel, ss_sel, hl_sel, te_sel, tye_sel, mrow_sel, We, be):
    return pl.pallas_call(
        _entries_body,
        out_shape=jax.ShapeDtypeStruct((NSEL, D), jnp.float32),
    )(seq_sel, ss_sel, hl_sel, te_sel, tye_sel, mrow_sel, We,
      be.reshape(1, D))


@functools.partial(jax.jit, static_argnums=())
def kernel(sequence, state_summary, holder_logits, z_per_step, W1, b1, W2, b2,
           Wp, bp, Wn, bn, type_embed, time_embed, We, be):
    seq2d = sequence.reshape(NROWS, D)
    etl2d, prev2d, next2d, nn3, zb3, ty3, hlp = _heads(
        seq2d, z_per_step.reshape(NROWS, NE), holder_logits.reshape(NROWS, NE),
        W1, b1.reshape(1, D), W2, b2.reshape(1, NT),
        Wp, bp.reshape(1, NE), Wn, bn.reshape(1, NE))

    (scores, mask_i, times, type_ids, idx_flat, idx_time, idx_ty,
     idx_msk) = _select(nn3, zb3, ty3)

    mrow_tbl = jnp.concatenate(
        [jnp.zeros((1, 128), jnp.float32), jnp.ones((1, 128), jnp.float32)])
    seq_sel, ss_sel, hl_sel, te_sel, tye_sel, mrow_sel = _sc_gather(
        seq2d, state_summary.reshape(NROWS, D), hlp, time_embed, type_embed,
        mrow_tbl, idx_flat, idx_time, idx_ty, idx_msk)

    ent2d = _entries(seq_sel, ss_sel, hl_sel, te_sel, tye_sel, mrow_sel,
                     We, be)

    return (ent2d.reshape(B, ME, D), mask_i, times,
            type_ids, etl2d.reshape(B, T, NT), prev2d.reshape(B, T, NE),
            next2d.reshape(B, T, NE), scores)


# speedup vs baseline: 1.0248x; 1.0248x over previous
import functools
import math

import jax
import jax.numpy as jnp
from jax import lax
from jax.experimental import pallas as pl
from jax.experimental.pallas import tpu as pltpu
from jax.experimental.pallas import tpu_sc as plsc

B, T, D = 4, 2048, 1024
NE = 32
NT = 7
ME = 32
THRESH = 0.4
ZBW = 0.15
MAX_TIME = 512

TT = 1024
NROWS = B * T
NSEL = B * ME
TPB = T // TT

_NEG_INF = float("-inf")


def _heads_body(x_ref, z_ref, hl_ref, w1_ref, b1_ref, w2_ref, b2_ref,
                wp_ref, bp_ref, wn_ref, bn_ref,
                etl_ref, prev_ref, next_ref, nn_ref, zb_ref, ty_ref, hlp_ref,
                w1b_ref):
    @pl.when(pl.program_id(0) == 0)
    def _():
        w1b_ref[...] = w1_ref[...].astype(jnp.bfloat16)

    x = x_ref[...]
    xb = x.astype(jnp.bfloat16)
    h = jnp.dot(xb, w1b_ref[...], preferred_element_type=jnp.float32) + b1_ref[...]
    g = 0.5 * h * (lax.erf(h / math.sqrt(2.0)) + 1.0)
    etl = jnp.dot(g.astype(jnp.bfloat16), w2_ref[...].astype(jnp.bfloat16),
                  preferred_element_type=jnp.float32) + b2_ref[...]
    etl_ref[...] = etl
    prev_ref[...] = jnp.dot(xb, wp_ref[...].astype(jnp.bfloat16),
                            preferred_element_type=jnp.float32) + bp_ref[...]
    next_ref[...] = jnp.dot(xb, wn_ref[...].astype(jnp.bfloat16),
                            preferred_element_type=jnp.float32) + bn_ref[...]

    emax = jnp.max(etl, axis=1, keepdims=True)
    ee = jnp.exp(etl - emax)
    esum = jnp.sum(ee, axis=1, keepdims=True)
    nn = 1.0 - ee[:, :1] / esum
    it2 = lax.broadcasted_iota(jnp.int32, (TT, NT), 1)
    ty = jnp.min(jnp.where(etl == emax, it2, NT), axis=1, keepdims=True)
    zb = jnp.max(jnp.abs(z_ref[...]), axis=1, keepdims=True)

    nn_ref[...] = nn.T.reshape(1, 1, TT)
    zb_ref[...] = zb.T.reshape(1, 1, TT)
    ty_ref[...] = ty.T.reshape(1, 1, TT)

    hl = hl_ref[...]
    hlp_ref[...] = jnp.concatenate(
        [hl, jnp.zeros((TT, 128 - NE), jnp.float32)], axis=1)


def _heads(seq2d, z2d, hl2d, W1, b1, W2, b2, Wp, bp, Wn, bn):
    nt = NROWS // TT
    row = lambda w: pl.BlockSpec((TT, w), lambda t: (t, 0))
    vec = lambda: pl.BlockSpec((1, 1, TT), lambda t: (t // TPB, 0, t % TPB))
    full = lambda a: pl.BlockSpec(a.shape, lambda t: (0,) * a.ndim)
    f32 = jnp.float32
    return pl.pallas_call(
        _heads_body,
        grid=(nt,),
        in_specs=[row(D), row(NE), row(NE)]
        + [full(a) for a in (W1, b1, W2, b2, Wp, bp, Wn, bn)],
        out_specs=[row(NT), row(NE), row(NE), vec(), vec(), vec(), row(128)],
        out_shape=[
            jax.ShapeDtypeStruct((NROWS, NT), f32),
            jax.ShapeDtypeStruct((NROWS, NE), f32),
            jax.ShapeDtypeStruct((NROWS, NE), f32),
            jax.ShapeDtypeStruct((B, 1, T), f32),
            jax.ShapeDtypeStruct((B, 1, T), f32),
            jax.ShapeDtypeStruct((B, 1, T), jnp.int32),
            jax.ShapeDtypeStruct((NROWS, 128), f32),
        ],
        scratch_shapes=[pltpu.VMEM((D, D), jnp.bfloat16)],
    )(seq2d, z2d, hl2d, W1, b1, W2, b2, Wp, bp, Wn, bn)


def _select_body(nn_ref, zbr_ref, ty_ref, scores_ref, mask_ref, times_ref,
                 tids_ref, idxflat_ref, idxtime_ref, idxty_ref, idxmsk_ref):
    nn = nn_ref[...].reshape(B, T)
    zb = zbr_ref[...].reshape(B, T)
    ty = ty_ref[...].reshape(B, T)

    zmax = jnp.max(zb, axis=1, keepdims=True)
    scores = nn + ZBW * zb / jnp.maximum(zmax, 1.0)
    scores_ref[...] = scores

    iota_t = lax.broadcasted_iota(jnp.int32, (B, T), 1)

    key0 = jnp.where(scores >= THRESH, scores, _NEG_INF)

    def ext_step(_, carry):
        key, selmask, cnt = carry
        m = jnp.max(key, axis=1, keepdims=True)
        has = m > _NEG_INF
        hit = key == m
        idx = jnp.min(jnp.where(hit, iota_t, T), axis=1, keepdims=True)
        pick = (iota_t == idx) & has
        return (jnp.where(pick, _NEG_INF, key), selmask | pick.astype(jnp.int32),
                cnt + has.astype(jnp.int32))

    selmask0 = jnp.zeros((B, T), jnp.int32)
    cnt0 = jnp.zeros((B, 1), jnp.int32)
    _, selmask, n_eff = lax.fori_loop(0, ME, ext_step, (key0, selmask0, cnt0))

    ikey = jnp.where(selmask > 0, iota_t, T)
    ch_cols, ty_cols = [], []
    for _ in range(ME):
        idx = jnp.min(ikey, axis=1, keepdims=True)
        ch_cols.append(idx)
        ty_cols.append(jnp.max(jnp.where(iota_t == idx, ty, 0),
                               axis=1, keepdims=True))
        ikey = jnp.where(iota_t == idx, T, ikey)
    ch = jnp.concatenate(ch_cols, axis=1)
    tysel = jnp.concatenate(ty_cols, axis=1)

    empty = n_eff == 0
    slot = lax.broadcasted_iota(jnp.int32, (B, ME), 1)
    ch = jnp.where(empty, slot, ch)
    tysel = jnp.where(empty, ty[:, :ME], tysel)
    n = jnp.where(empty, ME, n_eff)
    mask = slot < n
    mask_i = mask.astype(jnp.int32)

    mask_ref[...] = mask
    times_ref[...] = jnp.where(mask, ch, 0)
    tids_ref[...] = jnp.where(mask, tysel, 0)

    def flat_row(a):
        return jnp.concatenate([a[b:b + 1, :] for b in range(B)], axis=1)

    bi = lax.broadcasted_iota(jnp.int32, (B, ME), 0)
    idxflat_ref[...] = flat_row(bi * T + jnp.minimum(ch, T - 1))
    idxtime_ref[...] = flat_row(jnp.clip(ch, 0, MAX_TIME - 1))
    idxty_ref[...] = flat_row(tysel)
    idxmsk_ref[...] = flat_row(mask_i)


def _select(nn, zb, ty):
    i32 = jnp.int32
    return pl.pallas_call(
        _select_body,
        out_shape=[
            jax.ShapeDtypeStruct((B, T), jnp.float32),
            jax.ShapeDtypeStruct((B, ME), jnp.bool_),
            jax.ShapeDtypeStruct((B, ME), i32),
            jax.ShapeDtypeStruct((B, ME), i32),
            jax.ShapeDtypeStruct((1, NSEL), i32),
            jax.ShapeDtypeStruct((1, NSEL), i32),
            jax.ShapeDtypeStruct((1, NSEL), i32),
            jax.ShapeDtypeStruct((1, NSEL), i32),
        ],
    )(nn, zb, ty)


_NC = 2
_NWORK = 16
_RPW = NSEL // _NWORK


def _sc_gather_body(seq_hbm, ss_hbm, hl_hbm, te_hbm, tye_hbm, mrow_hbm,
                    idxf_hbm, idxt_hbm, idxy_hbm, idxm_hbm,
                    seqo_hbm, sso_hbm, hlo_hbm, teo_hbm, tyeo_hbm, mrowo_hbm,
                    ia_v, ib_v, ic_v, buf1_v, buf2_v, sbuf_v, sem):
    wid = lax.axis_index("s") * _NC + lax.axis_index("c")
    j = lax.rem(wid, _NWORK)
    base = j * _RPW

    @pl.when(wid < _NWORK)
    def _():
        pltpu.sync_copy(idxf_hbm.at[0, pl.ds(base, _RPW)], ia_v)
        pltpu.sync_copy(idxy_hbm.at[0, pl.ds(base, _RPW)], ic_v)
        c1 = pltpu.async_copy(seq_hbm.at[ia_v], buf1_v, sem)
        c2 = pltpu.async_copy(hl_hbm.at[ia_v], sbuf_v, sem)
        c3 = pltpu.async_copy(tye_hbm.at[ic_v], buf2_v, sem)
        c1.wait()
        c2.wait()
        c3.wait()
        o1 = pltpu.async_copy(buf1_v, seqo_hbm.at[pl.ds(base, _RPW)], sem)
        o2 = pltpu.async_copy(sbuf_v, hlo_hbm.at[pl.ds(base, _RPW)], sem)
        o3 = pltpu.async_copy(buf2_v, tyeo_hbm.at[pl.ds(base, _RPW)], sem)
        o1.wait()
        o2.wait()
        o3.wait()

    @pl.when(wid >= _NWORK)
    def _():
        pltpu.sync_copy(idxf_hbm.at[0, pl.ds(base, _RPW)], ia_v)
        pltpu.sync_copy(idxt_hbm.at[0, pl.ds(base, _RPW)], ib_v)
        pltpu.sync_copy(idxm_hbm.at[0, pl.ds(base, _RPW)], ic_v)
        c1 = pltpu.async_copy(ss_hbm.at[ia_v], buf1_v, sem)
        c2 = pltpu.async_copy(te_hbm.at[ib_v], buf2_v, sem)
        c3 = pltpu.async_copy(mrow_hbm.at[ic_v], sbuf_v, sem)
        c1.wait()
        c2.wait()
        c3.wait()
        o1 = pltpu.async_copy(buf1_v, sso_hbm.at[pl.ds(base, _RPW)], sem)
        o2 = pltpu.async_copy(buf2_v, teo_hbm.at[pl.ds(base, _RPW)], sem)
        o3 = pltpu.async_copy(sbuf_v, mrowo_hbm.at[pl.ds(base, _RPW)], sem)
        o1.wait()
        o2.wait()
        o3.wait()


def _sc_gather(seq2d, ss2d, hlp, time_embed, type_embed, mrow_tbl,
               idx_flat, idx_time, idx_ty, idx_msk):
    f32 = jnp.float32
    mesh = plsc.VectorSubcoreMesh(core_axis_name="c", subcore_axis_name="s")
    k = pl.kernel(
        _sc_gather_body,
        out_type=[
            jax.ShapeDtypeStruct((NSEL, D), f32),
            jax.ShapeDtypeStruct((NSEL, D), f32),
            jax.ShapeDtypeStruct((NSEL, 128), f32),
            jax.ShapeDtypeStruct((NSEL, D), f32),
            jax.ShapeDtypeStruct((NSEL, D), f32),
            jax.ShapeDtypeStruct((NSEL, 128), f32),
        ],
        mesh=mesh,
        scratch_types=[
            pltpu.VMEM((_RPW,), jnp.int32),
            pltpu.VMEM((_RPW,), jnp.int32),
            pltpu.VMEM((_RPW,), jnp.int32),
            pltpu.VMEM((_RPW, D), f32),
            pltpu.VMEM((_RPW, D), f32),
            pltpu.VMEM((_RPW, 128), f32),
            pltpu.SemaphoreType.DMA,
        ],
    )
    return k(seq2d, ss2d, hlp, time_embed, type_embed, mrow_tbl,
             idx_flat, idx_time, idx_ty, idx_msk)


def _entries_body(seqs_ref, sss_ref, hls_ref, tes_ref, tyes_ref, mrows_ref,
                  we_ref, be_ref, out_ref):
    hl = hls_ref[:, :NE]
    hmax = jnp.max(hl, axis=1, keepdims=True)
    he = jnp.exp(hl - hmax)
    hp = he / jnp.sum(he, axis=1, keepdims=True)

    acc = jnp.dot(seqs_ref[...].astype(jnp.bfloat16),
                  we_ref[:D].astype(jnp.bfloat16),
                  preferred_element_type=jnp.float32)
    acc = acc + jnp.dot(sss_ref[...].astype(jnp.bfloat16),
                        we_ref[D:2 * D].astype(jnp.bfloat16),
                        preferred_element_type=jnp.float32)
    acc = acc + jnp.dot(hp.astype(jnp.bfloat16),
                        we_ref[2 * D:].astype(jnp.bfloat16),
                        preferred_element_type=jnp.float32)
    acc = acc + be_ref[...] + tyes_ref[...] + tes_ref[...]
    out_ref[...] = acc * mrows_ref[:, :1]


def _entries(seq_sel, ss_sel, hl_sel, te_sel, tye_sel, mrow_sel, We, be):
    return pl.pallas_call(
        _entries_body,
        out_shape=jax.ShapeDtypeStruct((NSEL, D), jnp.float32),
    )(seq_sel, ss_sel, hl_sel, te_sel, tye_sel, mrow_sel, We,
      be.reshape(1, D))


@functools.partial(jax.jit, static_argnums=())
def kernel(sequence, state_summary, holder_logits, z_per_step, W1, b1, W2, b2,
           Wp, bp, Wn, bn, type_embed, time_embed, We, be):
    seq2d = sequence.reshape(NROWS, D)
    etl2d, prev2d, next2d, nn3, zb3, ty3, hlp = _heads(
        seq2d, z_per_step.reshape(NROWS, NE), holder_logits.reshape(NROWS, NE),
        W1, b1.reshape(1, D), W2, b2.reshape(1, NT),
        Wp, bp.reshape(1, NE), Wn, bn.reshape(1, NE))

    (scores, mask_i, times, type_ids, idx_flat, idx_time, idx_ty,
     idx_msk) = _select(nn3, zb3, ty3)

    mrow_tbl = jnp.concatenate(
        [jnp.zeros((1, 128), jnp.float32), jnp.ones((1, 128), jnp.float32)])
    seq_sel, ss_sel, hl_sel, te_sel, tye_sel, mrow_sel = _sc_gather(
        seq2d, state_summary.reshape(NROWS, D), hlp, time_embed, type_embed,
        mrow_tbl, idx_flat, idx_time, idx_ty, idx_msk)

    ent2d = _entries(seq_sel, ss_sel, hl_sel, te_sel, tye_sel, mrow_sel,
                     We, be)

    return (ent2d.reshape(B, ME, D), mask_i, times,
            type_ids, etl2d.reshape(B, T, NT), prev2d.reshape(B, T, NE),
            next2d.reshape(B, T, NE), scores)


# speedup vs baseline: 1.0468x; 1.0215x over previous
import functools
import math

import jax
import jax.numpy as jnp
from jax import lax
from jax.experimental import pallas as pl
from jax.experimental.pallas import tpu as pltpu
from jax.experimental.pallas import tpu_sc as plsc

B, T, D = 4, 2048, 1024
NE = 32
NT = 7
ME = 32
THRESH = 0.4
ZBW = 0.15
MAX_TIME = 512

TT = 1024
NROWS = B * T
NSEL = B * ME
NTILES = NROWS // TT
TPB = T // TT

_NEG_INF = float("-inf")


def _selection_tail(nn, zb, ty, scores_ref, mask_ref, times_ref, tids_ref,
                    idxflat_ref, idxtime_ref, idxty_ref, idxmsk_ref):
    zmax = jnp.max(zb, axis=1, keepdims=True)
    scores = nn + ZBW * zb / jnp.maximum(zmax, 1.0)
    scores_ref[...] = scores

    iota_t = lax.broadcasted_iota(jnp.int32, (B, T), 1)

    key0 = jnp.where(scores >= THRESH, scores, _NEG_INF)

    def ext_step(_, carry):
        key, selmask, cnt = carry
        m = jnp.max(key, axis=1, keepdims=True)
        has = m > _NEG_INF
        hit = key == m
        idx = jnp.min(jnp.where(hit, iota_t, T), axis=1, keepdims=True)
        pick = (iota_t == idx) & has
        return (jnp.where(pick, _NEG_INF, key), selmask | pick.astype(jnp.int32),
                cnt + has.astype(jnp.int32))

    selmask0 = jnp.zeros((B, T), jnp.int32)
    cnt0 = jnp.zeros((B, 1), jnp.int32)
    _, selmask, n_eff = lax.fori_loop(0, ME, ext_step, (key0, selmask0, cnt0))

    ikey = jnp.where(selmask > 0, iota_t, T)
    ch_cols, ty_cols = [], []
    for _ in range(ME):
        idx = jnp.min(ikey, axis=1, keepdims=True)
        ch_cols.append(idx)
        ty_cols.append(jnp.max(jnp.where(iota_t == idx, ty, 0),
                               axis=1, keepdims=True))
        ikey = jnp.where(iota_t == idx, T, ikey)
    ch = jnp.concatenate(ch_cols, axis=1)
    tysel = jnp.concatenate(ty_cols, axis=1)

    empty = n_eff == 0
    slot = lax.broadcasted_iota(jnp.int32, (B, ME), 1)
    ch = jnp.where(empty, slot, ch)
    tysel = jnp.where(empty, ty[:, :ME], tysel)
    n = jnp.where(empty, ME, n_eff)
    mask = slot < n
    mask_i = mask.astype(jnp.int32)

    mask_ref[...] = mask
    times_ref[...] = jnp.where(mask, ch, 0)
    tids_ref[...] = jnp.where(mask, tysel, 0)

    def flat_row(a):
        return jnp.concatenate([a[b:b + 1, :] for b in range(B)], axis=1)

    bi = lax.broadcasted_iota(jnp.int32, (B, ME), 0)
    idxflat_ref[...] = flat_row(bi * T + jnp.minimum(ch, T - 1))
    idxtime_ref[...] = flat_row(jnp.clip(ch, 0, MAX_TIME - 1))
    idxty_ref[...] = flat_row(tysel)
    idxmsk_ref[...] = flat_row(mask_i)


def _heads_body(x_ref, z_ref, hl_ref, w1_ref, b1_ref, w2_ref, b2_ref,
                wp_ref, bp_ref, wn_ref, bn_ref,
                etl_ref, prev_ref, next_ref, hlp_ref,
                scores_ref, mask_ref, times_ref, tids_ref,
                idxflat_ref, idxtime_ref, idxty_ref, idxmsk_ref,
                w1b_ref, nn_s, zb_s, ty_s):
    t = pl.program_id(0)

    @pl.when(t == 0)
    def _():
        w1b_ref[...] = w1_ref[...].astype(jnp.bfloat16)

    x = x_ref[...]
    xb = x.astype(jnp.bfloat16)
    h = jnp.dot(xb, w1b_ref[...], preferred_element_type=jnp.float32) + b1_ref[...]
    g = 0.5 * h * (lax.erf(h / math.sqrt(2.0)) + 1.0)
    etl = jnp.dot(g.astype(jnp.bfloat16), w2_ref[...].astype(jnp.bfloat16),
                  preferred_element_type=jnp.float32) + b2_ref[...]
    etl_ref[...] = etl
    prev_ref[...] = jnp.dot(xb, wp_ref[...].astype(jnp.bfloat16),
                            preferred_element_type=jnp.float32) + bp_ref[...]
    next_ref[...] = jnp.dot(xb, wn_ref[...].astype(jnp.bfloat16),
                            preferred_element_type=jnp.float32) + bn_ref[...]

    emax = jnp.max(etl, axis=1, keepdims=True)
    ee = jnp.exp(etl - emax)
    esum = jnp.sum(ee, axis=1, keepdims=True)
    nn = 1.0 - ee[:, :1] / esum
    it2 = lax.broadcasted_iota(jnp.int32, (TT, NT), 1)
    ty = jnp.min(jnp.where(etl == emax, it2, NT), axis=1, keepdims=True)
    zb = jnp.max(jnp.abs(z_ref[...]), axis=1, keepdims=True)

    nn_s[pl.ds(t, 1), :, :] = nn.T.reshape(1, 1, TT)
    zb_s[pl.ds(t, 1), :, :] = zb.T.reshape(1, 1, TT)
    ty_s[pl.ds(t, 1), :, :] = ty.T.reshape(1, 1, TT)

    hl = hl_ref[...]
    hlp_ref[...] = jnp.concatenate(
        [hl, jnp.zeros((TT, 128 - NE), jnp.float32)], axis=1)

    @pl.when(t == NTILES - 1)
    def _():
        def rows(s_ref):
            batches = []
            for b in range(B):
                parts = [s_ref[b * TPB + k, 0:1, :] for k in range(TPB)]
                batches.append(jnp.concatenate(parts, axis=1)
                               if TPB > 1 else parts[0])
            return jnp.concatenate(batches, axis=0)

        _selection_tail(rows(nn_s), rows(zb_s), rows(ty_s),
                        scores_ref, mask_ref, times_ref, tids_ref,
                        idxflat_ref, idxtime_ref, idxty_ref, idxmsk_ref)


def _heads(seq2d, z2d, hl2d, W1, b1, W2, b2, Wp, bp, Wn, bn):
    row = lambda w: pl.BlockSpec((TT, w), lambda t: (t, 0))
    full2 = lambda s: pl.BlockSpec(s, lambda t: (0,) * len(s))
    f32 = jnp.float32
    i32 = jnp.int32
    return pl.pallas_call(
        _heads_body,
        grid=(NTILES,),
        in_specs=[row(D), row(NE), row(NE)]
        + [full2(a.shape) for a in (W1, b1, W2, b2, Wp, bp, Wn, bn)],
        out_specs=[row(NT), row(NE), row(NE), row(128),
                   full2((B, T)), full2((B, ME)), full2((B, ME)),
                   full2((B, ME)), full2((1, NSEL)), full2((1, NSEL)),
                   full2((1, NSEL)), full2((1, NSEL))],
        out_shape=[
            jax.ShapeDtypeStruct((NROWS, NT), f32),
            jax.ShapeDtypeStruct((NROWS, NE), f32),
            jax.ShapeDtypeStruct((NROWS, NE), f32),
            jax.ShapeDtypeStruct((NROWS, 128), f32),
            jax.ShapeDtypeStruct((B, T), f32),
            jax.ShapeDtypeStruct((B, ME), jnp.bool_),
            jax.ShapeDtypeStruct((B, ME), i32),
            jax.ShapeDtypeStruct((B, ME), i32),
            jax.ShapeDtypeStruct((1, NSEL), i32),
            jax.ShapeDtypeStruct((1, NSEL), i32),
            jax.ShapeDtypeStruct((1, NSEL), i32),
            jax.ShapeDtypeStruct((1, NSEL), i32),
        ],
        scratch_shapes=[
            pltpu.VMEM((D, D), jnp.bfloat16),
            pltpu.VMEM((NTILES, 1, TT), f32),
            pltpu.VMEM((NTILES, 1, TT), f32),
            pltpu.VMEM((NTILES, 1, TT), i32),
        ],
    )(seq2d, z2d, hl2d, W1, b1, W2, b2, Wp, bp, Wn, bn)


_NC = 2
_NWORK = 16
_RPW = NSEL // _NWORK


def _sc_gather_body(seq_hbm, ss_hbm, hl_hbm, te_hbm, tye_hbm, mrow_hbm,
                    idxf_hbm, idxt_hbm, idxy_hbm, idxm_hbm,
                    seqo_hbm, sso_hbm, hlo_hbm, teo_hbm, tyeo_hbm, mrowo_hbm,
                    ia_v, ib_v, ic_v, buf1_v, buf2_v, sbuf_v, sem):
    wid = lax.axis_index("s") * _NC + lax.axis_index("c")
    j = lax.rem(wid, _NWORK)
    base = j * _RPW

    @pl.when(wid < _NWORK)
    def _():
        pltpu.sync_copy(idxf_hbm.at[0, pl.ds(base, _RPW)], ia_v)
        pltpu.sync_copy(idxy_hbm.at[0, pl.ds(base, _RPW)], ic_v)
        c1 = pltpu.async_copy(seq_hbm.at[ia_v], buf1_v, sem)
        c2 = pltpu.async_copy(hl_hbm.at[ia_v], sbuf_v, sem)
        c3 = pltpu.async_copy(tye_hbm.at[ic_v], buf2_v, sem)
        c1.wait()
        c2.wait()
        c3.wait()
        o1 = pltpu.async_copy(buf1_v, seqo_hbm.at[pl.ds(base, _RPW)], sem)
        o2 = pltpu.async_copy(sbuf_v, hlo_hbm.at[pl.ds(base, _RPW)], sem)
        o3 = pltpu.async_copy(buf2_v, tyeo_hbm.at[pl.ds(base, _RPW)], sem)
        o1.wait()
        o2.wait()
        o3.wait()

    @pl.when(wid >= _NWORK)
    def _():
        pltpu.sync_copy(idxf_hbm.at[0, pl.ds(base, _RPW)], ia_v)
        pltpu.sync_copy(idxt_hbm.at[0, pl.ds(base, _RPW)], ib_v)
        pltpu.sync_copy(idxm_hbm.at[0, pl.ds(base, _RPW)], ic_v)
        c1 = pltpu.async_copy(ss_hbm.at[ia_v], buf1_v, sem)
        c2 = pltpu.async_copy(te_hbm.at[ib_v], buf2_v, sem)
        c3 = pltpu.async_copy(mrow_hbm.at[ic_v], sbuf_v, sem)
        c1.wait()
        c2.wait()
        c3.wait()
        o1 = pltpu.async_copy(buf1_v, sso_hbm.at[pl.ds(base, _RPW)], sem)
        o2 = pltpu.async_copy(buf2_v, teo_hbm.at[pl.ds(base, _RPW)], sem)
        o3 = pltpu.async_copy(sbuf_v, mrowo_hbm.at[pl.ds(base, _RPW)], sem)
        o1.wait()
        o2.wait()
        o3.wait()


def _sc_gather(seq2d, ss2d, hlp, time_embed, type_embed, mrow_tbl,
               idx_flat, idx_time, idx_ty, idx_msk):
    f32 = jnp.float32
    mesh = plsc.VectorSubcoreMesh(core_axis_name="c", subcore_axis_name="s")
    k = pl.kernel(
        _sc_gather_body,
        out_type=[
            jax.ShapeDtypeStruct((NSEL, D), f32),
            jax.ShapeDtypeStruct((NSEL, D), f32),
            jax.ShapeDtypeStruct((NSEL, 128), f32),
            jax.ShapeDtypeStruct((NSEL, D), f32),
            jax.ShapeDtypeStruct((NSEL, D), f32),
            jax.ShapeDtypeStruct((NSEL, 128), f32),
        ],
        mesh=mesh,
        scratch_types=[
            pltpu.VMEM((_RPW,), jnp.int32),
            pltpu.VMEM((_RPW,), jnp.int32),
            pltpu.VMEM((_RPW,), jnp.int32),
            pltpu.VMEM((_RPW, D), f32),
            pltpu.VMEM((_RPW, D), f32),
            pltpu.VMEM((_RPW, 128), f32),
            pltpu.SemaphoreType.DMA,
        ],
    )
    return k(seq2d, ss2d, hlp, time_embed, type_embed, mrow_tbl,
             idx_flat, idx_time, idx_ty, idx_msk)


def _entries_body(seqs_ref, sss_ref, hls_ref, tes_ref, tyes_ref, mrows_ref,
                  we_ref, be_ref, out_ref):
    hl = hls_ref[:, :NE]
    hmax = jnp.max(hl, axis=1, keepdims=True)
    he = jnp.exp(hl - hmax)
    hp = he / jnp.sum(he, axis=1, keepdims=True)

    acc = jnp.dot(seqs_ref[...].astype(jnp.bfloat16),
                  we_ref[:D].astype(jnp.bfloat16),
                  preferred_element_type=jnp.float32)
    acc = acc + jnp.dot(sss_ref[...].astype(jnp.bfloat16),
                        we_ref[D:2 * D].astype(jnp.bfloat16),
                        preferred_element_type=jnp.float32)
    acc = acc + jnp.dot(hp.astype(jnp.bfloat16),
                        we_ref[2 * D:].astype(jnp.bfloat16),
                        preferred_element_type=jnp.float32)
    acc = acc + be_ref[...] + tyes_ref[...] + tes_ref[...]
    out_ref[...] = acc * mrows_ref[:, :1]


def _entries(seq_sel, ss_sel, hl_sel, te_sel, tye_sel, mrow_sel, We, be):
    return pl.pallas_call(
        _entries_body,
        out_shape=jax.ShapeDtypeStruct((NSEL, D), jnp.float32),
    )(seq_sel, ss_sel, hl_sel, te_sel, tye_sel, mrow_sel, We,
      be.reshape(1, D))


@functools.partial(jax.jit, static_argnums=())
def kernel(sequence, state_summary, holder_logits, z_per_step, W1, b1, W2, b2,
           Wp, bp, Wn, bn, type_embed, time_embed, We, be):
    seq2d = sequence.reshape(NROWS, D)
    (etl2d, prev2d, next2d, hlp, scores, mask_b, times, type_ids,
     idx_flat, idx_time, idx_ty, idx_msk) = _heads(
        seq2d, z_per_step.reshape(NROWS, NE), holder_logits.reshape(NROWS, NE),
        W1, b1.reshape(1, D), W2, b2.reshape(1, NT),
        Wp, bp.reshape(1, NE), Wn, bn.reshape(1, NE))

    mrow_tbl = jnp.concatenate(
        [jnp.zeros((1, 128), jnp.float32), jnp.ones((1, 128), jnp.float32)])
    seq_sel, ss_sel, hl_sel, te_sel, tye_sel, mrow_sel = _sc_gather(
        seq2d, state_summary.reshape(NROWS, D), hlp, time_embed, type_embed,
        mrow_tbl, idx_flat, idx_time, idx_ty, idx_msk)

    ent2d = _entries(seq_sel, ss_sel, hl_sel, te_sel, tye_sel, mrow_sel,
                     We, be)

    return (ent2d.reshape(B, ME, D), mask_b, times,
            type_ids, etl2d.reshape(B, T, NT), prev2d.reshape(B, T, NE),
            next2d.reshape(B, T, NE), scores)
